# baseline (speedup 1.0000x reference)
import jax
import jax.numpy as jnp
from jax import lax
from jax.experimental import pallas as pl
from jax.experimental.pallas import tpu as pltpu
from jax.experimental.pallas import tpu_sc as plsc

_EMBED = 128
_B = 4096
_L = 200
_REG = 0.01
_NW = 32
_UPW = _B // _NW
_RPW = _UPW * _L
_NCH = _EMBED // 16
_CH0, _CH1 = 104, 96
_NSQV = 4


def _mf_body(user_hbm, item_hbm, uemb_hbm, iemb_hbm, pred_hbm, l2_hbm,
             uidx_v, iidx_v, urows_v, rows_v, pred_v, sq_v, l2_v, sem0, sem1):
    c = lax.axis_index("c")
    s = lax.axis_index("s")
    wid = s * 2 + c
    ubase = pl.multiple_of(wid * _UPW, 8)
    rbase = pl.multiple_of(wid * _RPW, 8)
    sems = (sem0, sem1)

    pltpu.sync_copy(user_hbm.at[pl.ds(ubase, _UPW)], uidx_v)
    pltpu.sync_copy(item_hbm.at[pl.ds(rbase, _RPW)], iidx_v)
    pltpu.async_copy(uemb_hbm.at[uidx_v], urows_v, sem0).wait()

    def fire(b, buf):
        base = pl.multiple_of(b * _L, 8)
        pltpu.async_copy(
            iemb_hbm.at[iidx_v.at[pl.ds(base, _CH0)]],
            rows_v.at[buf, pl.ds(0, _CH0)], sems[buf])
        pltpu.async_copy(
            iemb_hbm.at[iidx_v.at[pl.ds(base + _CH0, _CH1)]],
            rows_v.at[buf, pl.ds(_CH0, _CH1)], sems[buf])

    def wait_rows(buf):
        pltpu.make_async_copy(
            iemb_hbm.at[pl.ds(0, _L)], rows_v.at[buf], sems[buf]).wait()

    zeros = jnp.zeros((16,), jnp.float32)
    lane = lax.iota(jnp.int32, 16)
    last_lane = lane == 15

    for j in range(_NCH - _NSQV, _NCH):
        sq_v[pl.ds(16 * (j - _NCH + _NSQV), 16)] = zeros

    def group16(u, sqs, buf, li0, pbase, sq_from):
        nsqs = list(sqs)
        for i in range(16):
            li = li0 + i
            acc = None
            for j in range(_NCH):
                r = rows_v[buf, li, pl.ds(16 * j, 16)]
                acc = u[j] * r if acc is None else acc + u[j] * r
                if i >= sq_from:
                    if j < _NCH - _NSQV:
                        nsqs[j] = nsqs[j] + r * r
                    else:
                        plsc.addupdate(
                            sq_v.at[pl.ds(16 * (j - _NCH + _NSQV), 16)],
                            r * r)
            c = jnp.cumsum(acc)
            plsc.store_compressed(
                pred_v.at[pl.ds(pbase + i, 16)], c, mask=last_lane)
        return tuple(nsqs)

    def pair_body(p, carry):
        sqs, usq = carry
        for q in (0, 1):
            b = p * 2 + q
            if q == 0:
                fire(b + 1, 1)
            else:
                @pl.when(p < _UPW // 2 - 1)
                def _():
                    fire(b + 1, 0)
            wait_rows(q)
            base = pl.multiple_of(b * _L, 8)

            u = [urows_v[b, pl.ds(16 * j, 16)] for j in range(_NCH)]
            for j in range(_NCH):
                usq = usq + u[j] * u[j]

            def group_body(g, isqs, u=u, q=q, base=base):
                return group16(u, isqs, q, g * 16, base + g * 16, 0)

            sqs = lax.fori_loop(0, (_L // 16), group_body, sqs)
            sqs = group16(u, sqs, q, _L - 16, base + _L - 16, 8)
        return (sqs, usq)

    fire(0, 0)
    init = (tuple(zeros for _ in range(_NCH)), zeros)
    sqs, usq = lax.fori_loop(0, _UPW // 2, pair_body, init)

    tot = usq * float(_L)
    for j in range(_NCH):
        tot = tot + sqs[j]
    for j in range(_NSQV):
        tot = tot + sq_v[pl.ds(16 * j, 16)]
    l2_v[...] = tot
    pltpu.sync_copy(pred_v.at[pl.ds(0, _RPW)], pred_hbm.at[pl.ds(rbase, _RPW)])
    pltpu.sync_copy(l2_v, l2_hbm.at[wid])


def kernel(user, item, user_embedding, item_embedding):
    user_flat = user.reshape(-1).astype(jnp.int32)
    item_flat = item.reshape(-1).astype(jnp.int32)
    mesh = plsc.VectorSubcoreMesh(core_axis_name="c", subcore_axis_name="s")
    pred_flat, l2p = pl.kernel(
        _mf_body,
        mesh=mesh,
        compiler_params=pltpu.CompilerParams(needs_layout_passes=False),
        out_type=[
            jax.ShapeDtypeStruct((_B * _L,), jnp.float32),
            jax.ShapeDtypeStruct((_NW, 16), jnp.float32),
        ],
        scratch_types=[
            pltpu.VMEM((_UPW,), jnp.int32),
            pltpu.VMEM((_RPW,), jnp.int32),
            pltpu.VMEM((_UPW, _EMBED), jnp.float32),
            pltpu.VMEM((2, _L, _EMBED), jnp.float32),
            pltpu.VMEM((_RPW + 16,), jnp.float32),
            pltpu.VMEM((16 * _NSQV,), jnp.float32),
            pltpu.VMEM((16,), jnp.float32),
            pltpu.SemaphoreType.DMA,
            pltpu.SemaphoreType.DMA,
        ],
    )(user_flat, item_flat, user_embedding, item_embedding)
    pred = pred_flat.reshape(_B, _L)
    l2 = _REG * jnp.sum(l2p)
    return (pred, l2)


# speedup vs baseline: 2.9792x; 2.9792x over previous
import jax
import jax.numpy as jnp
from jax import lax
from jax.experimental import pallas as pl
from jax.experimental.pallas import tpu as pltpu
from jax.experimental.pallas import tpu_sc as plsc

_EMBED = 128
_B = 4096
_L = 200
_REG = 0.01
_NW = 32
_UPW = _B // _NW
_RPW = _UPW * _L
_NCH = _EMBED // 16
_CH0, _CH1 = 104, 96
_NSQV = 4


def _mf_body(user_hbm, item_hbm, uemb_hbm, iemb_hbm, pred_hbm, l2_hbm,
             uidx_v, iidx_v, urows_v, rows_v, pred_v, l2_v, sem0, sem1):
    c = lax.axis_index("c")
    s = lax.axis_index("s")
    wid = s * 2 + c
    ubase = pl.multiple_of(wid * _UPW, 8)
    rbase = pl.multiple_of(wid * _RPW, 8)
    sems = (sem0, sem1)

    pltpu.sync_copy(user_hbm.at[pl.ds(ubase, _UPW)], uidx_v)
    pltpu.sync_copy(item_hbm.at[pl.ds(rbase, _RPW)], iidx_v)
    pltpu.async_copy(uemb_hbm.at[uidx_v], urows_v, sem0).wait()

    def fire(b, buf):
        base = pl.multiple_of(b * _L, 8)
        pltpu.async_copy(
            iemb_hbm.at[iidx_v.at[pl.ds(base, _CH0)]],
            rows_v.at[buf, pl.ds(0, _CH0)], sems[buf])
        pltpu.async_copy(
            iemb_hbm.at[iidx_v.at[pl.ds(base + _CH0, _CH1)]],
            rows_v.at[buf, pl.ds(_CH0, _CH1)], sems[buf])

    def wait_rows(buf):
        pltpu.make_async_copy(
            iemb_hbm.at[pl.ds(0, _L)], rows_v.at[buf], sems[buf]).wait()

    zeros = jnp.zeros((16,), jnp.float32)
    lane = lax.iota(jnp.int32, 16)
    lane_eq = [lane == i for i in range(16)]

    def group16(u, sqs, buf, li0, pbase, sq_from):
        nsqs = list(sqs)
        psum = zeros
        for i in range(16):
            li = li0 + i
            acc = None
            for j in range(_NCH):
                r = rows_v[buf, li, pl.ds(16 * j, 16)]
                acc = u[j] * r if acc is None else acc + u[j] * r
                if i >= sq_from:
                    nsqs[j] = nsqs[j] + r * r
            psum = jnp.where(lane_eq[i], jnp.sum(acc), psum)
        pred_v[pl.ds(pbase, 16)] = psum
        return tuple(nsqs)

    def pair_body(p, carry):
        sqs, usq = carry
        for q in (0, 1):
            b = p * 2 + q
            if q == 0:
                fire(b + 1, 1)
            else:
                @pl.when(p < _UPW // 2 - 1)
                def _():
                    fire(b + 1, 0)
            wait_rows(q)
            base = pl.multiple_of(b * _L, 8)

            u = [urows_v[b, pl.ds(16 * j, 16)] for j in range(_NCH)]
            for j in range(_NCH):
                usq = usq + u[j] * u[j]

            def group_body(g, isqs, u=u, q=q, base=base):
                return group16(u, isqs, q, g * 16, base + g * 16, 0)

            sqs = lax.fori_loop(0, (_L // 16), group_body, sqs)
            sqs = group16(u, sqs, q, _L - 16, base + _L - 16, 8)
        return (sqs, usq)

    fire(0, 0)
    init = (tuple(zeros for _ in range(_NCH)), zeros)
    sqs, usq = lax.fori_loop(0, _UPW // 2, pair_body, init)

    tot = usq * float(_L)
    for j in range(_NCH):
        tot = tot + sqs[j]
    l2_v[...] = tot
    pltpu.sync_copy(pred_v.at[pl.ds(0, _RPW)], pred_hbm.at[pl.ds(rbase, _RPW)])
    pltpu.sync_copy(l2_v, l2_hbm.at[wid])


def kernel(user, item, user_embedding, item_embedding):
    user_flat = user.reshape(-1).astype(jnp.int32)
    item_flat = item.reshape(-1).astype(jnp.int32)
    mesh = plsc.VectorSubcoreMesh(core_axis_name="c", subcore_axis_name="s")
    pred_flat, l2p = pl.kernel(
        _mf_body,
        mesh=mesh,
        compiler_params=pltpu.CompilerParams(needs_layout_passes=False),
        out_type=[
            jax.ShapeDtypeStruct((_B * _L,), jnp.float32),
            jax.ShapeDtypeStruct((_NW, 16), jnp.float32),
        ],
        scratch_types=[
            pltpu.VMEM((_UPW,), jnp.int32),
            pltpu.VMEM((_RPW,), jnp.int32),
            pltpu.VMEM((_UPW, _EMBED), jnp.float32),
            pltpu.VMEM((2, _L, _EMBED), jnp.float32),
            pltpu.VMEM((_RPW + 16,), jnp.float32),
            pltpu.VMEM((16,), jnp.float32),
            pltpu.SemaphoreType.DMA,
            pltpu.SemaphoreType.DMA,
        ],
    )(user_flat, item_flat, user_embedding, item_embedding)
    pred = pred_flat.reshape(_B, _L)
    l2 = _REG * jnp.sum(l2p)
    return (pred, l2)


# speedup vs baseline: 2.9954x; 1.0054x over previous
import jax
import jax.numpy as jnp
from jax import lax
from jax.experimental import pallas as pl
from jax.experimental.pallas import tpu as pltpu
from jax.experimental.pallas import tpu_sc as plsc

_EMBED = 128
_B = 4096
_L = 200
_REG = 0.01
_NW = 32
_UPW = _B // _NW
_NCH = _EMBED // 16
_CH0, _CH1 = 104, 96


def _mf_body(user_hbm, item_hbm, uemb_hbm, iemb_hbm, pred_hbm, l2_hbm,
             uidx_v, iidx_v, iflat_v, urows_v, rows_v, pred_v, l2_v,
             sem0, sem1):
    c = lax.axis_index("c")
    s = lax.axis_index("s")
    wid = s * 2 + c
    ubase = pl.multiple_of(wid * _UPW, 8)
    sems = (sem0, sem1)

    pltpu.sync_copy(item_hbm.at[pl.ds(ubase, _UPW)], iidx_v)
    pltpu.sync_copy(user_hbm.at[pl.ds(ubase, _UPW)], uidx_v)
    pltpu.async_copy(uemb_hbm.at[uidx_v], urows_v, sem0).wait()

    def prep_idx(b, buf):
        off = buf * (_L + 8)
        for k in list(range(0, _L - 16, 16)) + [_L - 16]:
            iflat_v[pl.ds(off + k, 16)] = iidx_v[b, pl.ds(k, 16)]

    def fire(b, buf):
        off = buf * (_L + 8)
        pltpu.async_copy(
            iemb_hbm.at[iflat_v.at[pl.ds(off, _CH0)]],
            rows_v.at[buf, pl.ds(0, _CH0)], sems[buf])
        pltpu.async_copy(
            iemb_hbm.at[iflat_v.at[pl.ds(off + _CH0, _CH1)]],
            rows_v.at[buf, pl.ds(_CH0, _CH1)], sems[buf])

    def wait_rows(buf):
        pltpu.make_async_copy(
            iemb_hbm.at[pl.ds(0, _L)], rows_v.at[buf], sems[buf]).wait()

    zeros = jnp.zeros((16,), jnp.float32)
    lane = lax.iota(jnp.int32, 16)
    lane_eq = [lane == i for i in range(16)]

    def group16(u, sqs, buf, ub, li0, sq_from):
        nsqs = list(sqs)
        psum = zeros
        for i in range(16):
            li = li0 + i
            acc = None
            for j in range(_NCH):
                r = rows_v[buf, li, pl.ds(16 * j, 16)]
                acc = u[j] * r if acc is None else acc + u[j] * r
                if i >= sq_from:
                    nsqs[j] = nsqs[j] + r * r
            psum = jnp.where(lane_eq[i], jnp.sum(acc), psum)
        pred_v[pl.ds(ub * _L + li0, 16)] = psum
        return tuple(nsqs)

    def pair_body(p, carry):
        sqs, usq = carry
        for q in (0, 1):
            b = p * 2 + q
            if q == 0:
                prep_idx(b + 1, 1)
                fire(b + 1, 1)
            else:
                @pl.when(p < _UPW // 2 - 1)
                def _():
                    prep_idx(b + 1, 0)
                    fire(b + 1, 0)
            wait_rows(q)

            u = [urows_v[b, pl.ds(16 * j, 16)] for j in range(_NCH)]
            for j in range(_NCH):
                usq = usq + u[j] * u[j]

            def group_body(g, isqs, u=u, q=q, b=b):
                return group16(u, isqs, q, b, g * 16, 0)

            sqs = lax.fori_loop(0, (_L // 16), group_body, sqs)
            sqs = group16(u, sqs, q, b, _L - 16, 8)
        return (sqs, usq)

    prep_idx(0, 0)
    fire(0, 0)
    init = (tuple(zeros for _ in range(_NCH)), zeros)
    sqs, usq = lax.fori_loop(0, _UPW // 2, pair_body, init)

    tot = usq * float(_L)
    for j in range(_NCH):
        tot = tot + sqs[j]
    l2_v[...] = tot
    pltpu.sync_copy(l2_v, l2_hbm.at[wid])
    rbase = pl.multiple_of(wid * (_UPW * _L), 8)
    pltpu.sync_copy(pred_v, pred_hbm.at[pl.ds(rbase, _UPW * _L)])


def kernel(user, item, user_embedding, item_embedding):
    user_flat = user.reshape(-1).astype(jnp.int32)
    mesh = plsc.VectorSubcoreMesh(core_axis_name="c", subcore_axis_name="s")
    pred, l2p = pl.kernel(
        _mf_body,
        mesh=mesh,
        compiler_params=pltpu.CompilerParams(needs_layout_passes=False),
        out_type=[
            jax.ShapeDtypeStruct((_B * _L,), jnp.float32),
            jax.ShapeDtypeStruct((_NW, 16), jnp.float32),
        ],
        scratch_types=[
            pltpu.VMEM((_UPW,), jnp.int32),
            pltpu.VMEM((_UPW, _L), jnp.int32),
            pltpu.VMEM((2 * (_L + 8),), jnp.int32),
            pltpu.VMEM((_UPW, _EMBED), jnp.float32),
            pltpu.VMEM((2, _L, _EMBED), jnp.float32),
            pltpu.VMEM((_UPW * _L,), jnp.float32),
            pltpu.VMEM((16,), jnp.float32),
            pltpu.SemaphoreType.DMA,
            pltpu.SemaphoreType.DMA,
        ],
    )(user_flat, item, user_embedding, item_embedding)
    l2 = _REG * jnp.sum(l2p)
    return (pred.reshape(_B, _L), l2)
